# Initial kernel scaffold; baseline (speedup 1.0000x reference)
import jax
import jax.numpy as jnp
from jax.experimental import pallas as pl


def kernel(node_features, edge_index, edge_type, src_idx, rel_idx, dst_idx, W1, W1_self, b1, W2, W2_self, b2, Rdiag):
    raise NotImplementedError("write your pallas kernel here")


# speedup vs baseline: 13.3989x; 13.3989x over previous
import functools

import jax
import jax.numpy as jnp
from jax import lax
from jax.experimental import pallas as pl
from jax.experimental.pallas import tpu as pltpu
from jax.experimental.pallas import tpu_sc as plsc

N = 10000
E = 320000
R = 4
D_IN = 128
D_MID = 64
D_LAT = 128
T = 100000

NC = 2
NS = 16
L = 16
NW = NC * NS

EPW = E // NW
ECH = 80
NCH = EPW // ECH

RN = R * N
RNP = 49152
DEG_SL = RNP // NS

NP = 10112
NP_SL = NP // NS

TPW = 3136
TCH = 112
TNCH = TPW // TCH
TP = TPW * NW

_mesh = functools.partial(
    plsc.VectorSubcoreMesh,
    core_axis_name="c", subcore_axis_name="s",
    num_cores=NC, num_subcores=NS)


def _prep_edges_body(src, dst, et, fs, fd):
    fs[...] = et[...] * N + src[...]
    fd[...] = et[...] * N + dst[...]


def _prep_triples_body(srcp, relp, dstp, fsr, fdr):
    fsr[...] = relp[...] * N + srcp[...]
    fdr[...] = dstp[...] + R * N


def _mm_body(x, w, o):
    o[0] = jnp.dot(x[...], w[0], preferred_element_type=jnp.float32)


def _matmul(h, Wstk, d_in, d_out):
    nb = 10
    bs = N // nb
    return pl.pallas_call(
        _mm_body,
        grid=(R + 1, nb),
        in_specs=[
            pl.BlockSpec((bs, d_in), lambda r, i: (i, 0)),
            pl.BlockSpec((1, d_in, d_out), lambda r, i: (r, 0, 0)),
        ],
        out_specs=pl.BlockSpec((1, bs, d_out), lambda r, i: (r, i, 0)),
        out_shape=jax.ShapeDtypeStruct((R + 1, N, d_out), jnp.float32),
    )(h, Wstk)


def _deginv_body(d, o):
    o[...] = 1.0 / jnp.maximum(d[0] + d[1], 1.0)


def _finish1_body(a0, a1, slf, b, o):
    o[...] = jnp.maximum(a0[...] + a1[...] + slf[...] + b[...], 0.0)


def _finish2_body(a0, a1, slf, b, rd, o):
    h2 = a0[...] + a1[...] + slf[...] + b[...]
    o[R] = h2
    for r in range(R):
        o[r] = h2 * rd[r]


def _deg_body(fd_hbm, zeros_hbm, out_hbm, fd_v, ones_v, deg_sh):
    c = lax.axis_index("c")
    s = lax.axis_index("s")
    wid = c * NS + s
    pltpu.sync_copy(zeros_hbm.at[pl.ds(s * DEG_SL, DEG_SL)],
                    deg_sh.at[pl.ds(s * DEG_SL, DEG_SL)])
    for g in range(ECH // L):
        ones_v[pl.ds(g * L, L)] = jnp.ones((L,), jnp.float32)
    pltpu.sync_copy(fd_hbm.at[wid], fd_v)
    plsc.subcore_barrier()

    def body(j, carry):
        pltpu.sync_copy(ones_v, deg_sh.at[fd_v.at[j]], add=True)
        return carry

    lax.fori_loop(0, NCH, body, 0)
    plsc.subcore_barrier()
    pltpu.sync_copy(deg_sh.at[pl.ds(s * DEG_SL, DEG_SL)],
                    out_hbm.at[pl.ds(c * RNP + s * DEG_SL, DEG_SL)])


def _deg_kernel(fd, zeros_deg):
    k = pl.kernel(
        _deg_body,
        out_type=jax.ShapeDtypeStruct((NC * RNP,), jnp.float32),
        mesh=_mesh(),
        compiler_params=pltpu.CompilerParams(
            needs_layout_passes=False, use_tc_tiling_on_sc=False),
        scratch_types=[
            pltpu.VMEM((NCH, ECH), jnp.int32),
            pltpu.VMEM((ECH,), jnp.float32),
            pltpu.VMEM_SHARED((RNP,), jnp.float32),
        ],
    )
    return k(fd, zeros_deg)


def _layer_body(D, compute_norm, *refs):
    if compute_norm:
        (table_hbm, fs_hbm, dst_hbm, fd_hbm, dinv_hbm, zeros_hbm,
         out_hbm, norm_out_hbm,
         fs_v, dst_v, norm_v, fd_v, rows_v, dinv_sh, agg_sh) = refs
    else:
        (table_hbm, fs_hbm, dst_hbm, norm_hbm, zeros_hbm,
         out_hbm,
         fs_v, dst_v, norm_c, rows_v, agg_sh) = refs
    c = lax.axis_index("c")
    s = lax.axis_index("s")
    wid = c * NS + s
    pltpu.sync_copy(zeros_hbm.at[pl.ds(s * NP_SL, NP_SL)],
                    agg_sh.at[pl.ds(s * NP_SL, NP_SL)])
    pltpu.sync_copy(fs_hbm.at[wid], fs_v)
    pltpu.sync_copy(dst_hbm.at[wid], dst_v)
    if compute_norm:
        pltpu.sync_copy(fd_hbm.at[wid], fd_v)
        pltpu.sync_copy(dinv_hbm.at[pl.ds(s * DEG_SL, DEG_SL)],
                        dinv_sh.at[pl.ds(s * DEG_SL, DEG_SL)])
        plsc.subcore_barrier()

        def nbody(j, carry):
            pltpu.sync_copy(dinv_sh.at[fd_v.at[j]], norm_v.at[j])
            return carry

        lax.fori_loop(0, NCH, nbody, 0)
        pltpu.sync_copy(norm_v, norm_out_hbm.at[wid])
    plsc.subcore_barrier()

    def ebody(j, carry):
        if not compute_norm:
            pltpu.sync_copy(norm_hbm.at[wid].at[j], norm_c)
        pltpu.sync_copy(table_hbm.at[fs_v.at[j]], rows_v)

        def rbody(i, icarry):
            if compute_norm:
                sv = plsc.load_gather(
                    norm_v, [jnp.full((L,), j, jnp.int32),
                             jnp.full((L,), i, jnp.int32)])
            else:
                sv = plsc.load_gather(
                    norm_c, [jnp.full((L,), i, jnp.int32)])
            for g in range(D // L):
                rows_v[i, pl.ds(g * L, L)] = (
                    rows_v[i, pl.ds(g * L, L)] * sv)
            return icarry

        lax.fori_loop(0, ECH, rbody, 0)
        pltpu.sync_copy(rows_v, agg_sh.at[dst_v.at[j]], add=True)
        return carry

    lax.fori_loop(0, NCH, ebody, 0)
    plsc.subcore_barrier()
    pltpu.sync_copy(agg_sh.at[pl.ds(s * NP_SL, NP_SL)],
                    out_hbm.at[c, pl.ds(s * NP_SL, NP_SL)])


def _layer1_kernel(table, fs, dst2, fd, dinv, zeros_np):
    k = pl.kernel(
        functools.partial(_layer_body, D_MID, True),
        out_type=(
            jax.ShapeDtypeStruct((NC, NP, D_MID), jnp.float32),
            jax.ShapeDtypeStruct((NW, NCH, ECH), jnp.float32),
        ),
        mesh=_mesh(),
        compiler_params=pltpu.CompilerParams(
            needs_layout_passes=False, use_tc_tiling_on_sc=False),
        scratch_types=[
            pltpu.VMEM((NCH, ECH), jnp.int32),
            pltpu.VMEM((NCH, ECH), jnp.int32),
            pltpu.VMEM((NCH, ECH), jnp.float32),
            pltpu.VMEM((NCH, ECH), jnp.int32),
            pltpu.VMEM((ECH, D_MID), jnp.float32),
            pltpu.VMEM_SHARED((RNP,), jnp.float32),
            pltpu.VMEM_SHARED((NP, D_MID), jnp.float32),
        ],
    )
    return k(table, fs, dst2, fd, dinv, zeros_np)


def _layer2_kernel(table, fs, dst2, norm, zeros_np):
    k = pl.kernel(
        functools.partial(_layer_body, D_LAT, False),
        out_type=jax.ShapeDtypeStruct((NC, NP, D_LAT), jnp.float32),
        mesh=_mesh(),
        compiler_params=pltpu.CompilerParams(needs_layout_passes=False),
        scratch_types=[
            pltpu.VMEM((NCH, ECH), jnp.int32),
            pltpu.VMEM((NCH, ECH), jnp.int32),
            pltpu.VMEM((ECH,), jnp.float32),
            pltpu.VMEM((ECH, D_LAT), jnp.float32),
            pltpu.VMEM_SHARED((NP, D_LAT), jnp.float32),
        ],
    )
    return k(table, fs, dst2, norm, zeros_np)


def _decoder_body(table_hbm, fsr_hbm, fdr_hbm, out_hbm,
                  fsr_v, fdr_v, srows_v, drows_v, scores_v):
    c = lax.axis_index("c")
    s = lax.axis_index("s")
    wid = c * NS + s
    pltpu.sync_copy(fsr_hbm.at[wid], fsr_v)
    pltpu.sync_copy(fdr_hbm.at[wid], fdr_v)

    def jbody(j, carry):
        pltpu.sync_copy(table_hbm.at[fsr_v.at[j]], srows_v)
        pltpu.sync_copy(table_hbm.at[fdr_v.at[j]], drows_v)

        for tg in range(TCH // L):
            tv = jnp.arange(L, dtype=jnp.int32) + (tg * L)

            def dbody(d8, acc):
                for u in range(8):
                    dv = jnp.full((L,), d8 * 8 + u, jnp.int32)
                    sc = plsc.load_gather(srows_v, [tv, dv])
                    dc = plsc.load_gather(drows_v, [tv, dv])
                    acc = acc + sc * dc
                return acc

            acc = lax.fori_loop(0, D_LAT // 8, dbody,
                                jnp.zeros((L,), jnp.float32))
            scores_v[j, pl.ds(tg * L, L)] = acc
        return carry

    lax.fori_loop(0, TNCH, jbody, 0)
    pltpu.sync_copy(scores_v, out_hbm.at[wid])


def _decoder_kernel(table, fsr, fdr):
    k = pl.kernel(
        _decoder_body,
        out_type=jax.ShapeDtypeStruct((NW, TNCH, TCH), jnp.float32),
        mesh=_mesh(),
        compiler_params=pltpu.CompilerParams(needs_layout_passes=False),
        scratch_types=[
            pltpu.VMEM((TNCH, TCH), jnp.int32),
            pltpu.VMEM((TNCH, TCH), jnp.int32),
            pltpu.VMEM((TCH, D_LAT), jnp.float32),
            pltpu.VMEM((TCH, D_LAT), jnp.float32),
            pltpu.VMEM((TNCH, TCH), jnp.float32),
        ],
    )
    return k(table, fsr, fdr)


def kernel(node_features, edge_index, edge_type, src_idx, rel_idx, dst_idx,
           W1, W1_self, b1, W2, W2_self, b2, Rdiag):
    f32 = jnp.float32
    src = edge_index[0].reshape(2500, 128)
    dst = edge_index[1].reshape(2500, 128)
    et = edge_type.reshape(2500, 128)

    fs, fd = pl.pallas_call(
        _prep_edges_body,
        out_shape=(jax.ShapeDtypeStruct((2500, 128), jnp.int32),
                   jax.ShapeDtypeStruct((2500, 128), jnp.int32)),
    )(src, dst, et)
    fs = fs.reshape(NW, NCH, ECH)
    fd = fd.reshape(NW, NCH, ECH)
    dst2 = edge_index[1].reshape(NW, NCH, ECH)

    padn = TP - T
    pad_iota = (jnp.arange(padn, dtype=jnp.int32) * 37) % N
    srcp = jnp.concatenate([src_idx, pad_iota]).reshape(TP // 128, 128)
    relp = jnp.concatenate(
        [rel_idx, jnp.zeros((padn,), jnp.int32)]).reshape(TP // 128, 128)
    dstp = jnp.concatenate([dst_idx, pad_iota]).reshape(TP // 128, 128)
    fsr, fdr = pl.pallas_call(
        _prep_triples_body,
        out_shape=(jax.ShapeDtypeStruct((TP // 128, 128), jnp.int32),
                   jax.ShapeDtypeStruct((TP // 128, 128), jnp.int32)),
    )(srcp, relp, dstp)
    fsr = fsr.reshape(NW, TNCH, TCH)
    fdr = fdr.reshape(NW, TNCH, TCH)

    zeros_deg = jnp.zeros((RNP,), f32)
    zeros_n1 = jnp.zeros((NP, D_MID), f32)
    zeros_n2 = jnp.zeros((NP, D_LAT), f32)

    deg_parts = _deg_kernel(fd, zeros_deg)
    dinv = pl.pallas_call(
        _deginv_body,
        out_shape=jax.ShapeDtypeStruct((RNP // 128, 128), f32),
    )(deg_parts.reshape(NC, RNP // 128, 128))
    dinv = dinv.reshape(RNP)

    W1a = jnp.concatenate([W1, W1_self[None]], axis=0)
    T1 = _matmul(node_features, W1a, D_IN, D_MID)
    table1 = T1.reshape((R + 1) * N, D_MID)
    agg1, norm = _layer1_kernel(table1, fs, dst2, fd, dinv, zeros_n1)

    nb = 10
    bs = N // nb
    H1 = pl.pallas_call(
        _finish1_body,
        grid=(nb,),
        in_specs=[
            pl.BlockSpec((bs, D_MID), lambda i: (i, 0)),
            pl.BlockSpec((bs, D_MID), lambda i: (i, 0)),
            pl.BlockSpec((bs, D_MID), lambda i: (i, 0)),
            pl.BlockSpec((1, D_MID), lambda i: (0, 0)),
        ],
        out_specs=pl.BlockSpec((bs, D_MID), lambda i: (i, 0)),
        out_shape=jax.ShapeDtypeStruct((N, D_MID), f32),
    )(agg1[0, :N], agg1[1, :N], T1[R], b1.reshape(1, D_MID))

    W2a = jnp.concatenate([W2, W2_self[None]], axis=0)
    T2 = _matmul(H1, W2a, D_MID, D_LAT)
    table2 = T2.reshape((R + 1) * N, D_LAT)
    agg2 = _layer2_kernel(table2, fs, dst2, norm, zeros_n2)

    tab_dec = pl.pallas_call(
        _finish2_body,
        grid=(nb,),
        in_specs=[
            pl.BlockSpec((bs, D_LAT), lambda i: (i, 0)),
            pl.BlockSpec((bs, D_LAT), lambda i: (i, 0)),
            pl.BlockSpec((bs, D_LAT), lambda i: (i, 0)),
            pl.BlockSpec((1, D_LAT), lambda i: (0, 0)),
            pl.BlockSpec((R, D_LAT), lambda i: (0, 0)),
        ],
        out_specs=pl.BlockSpec((R + 1, bs, D_LAT), lambda i: (0, i, 0)),
        out_shape=jax.ShapeDtypeStruct((R + 1, N, D_LAT), f32),
    )(agg2[0, :N], agg2[1, :N], T2[R], b2.reshape(1, D_LAT), Rdiag)

    scores = _decoder_kernel(tab_dec.reshape((R + 1) * N, D_LAT), fsr, fdr)
    return scores.reshape(TP)[:T]


# speedup vs baseline: 17.1502x; 1.2800x over previous
import functools

import jax
import jax.numpy as jnp
from jax import lax
from jax.experimental import pallas as pl
from jax.experimental.pallas import tpu as pltpu
from jax.experimental.pallas import tpu_sc as plsc

N = 10000
E = 320000
R = 4
D_IN = 128
D_MID = 64
D_LAT = 128
T = 100000

NC = 2
NS = 16
L = 16
NW = NC * NS

EPW = E // NW
ECH = 80
NCH = EPW // ECH

RN = R * N
RNP = 49152
DEG_SL = RNP // NS

NP = 10112
NP_SL = NP // NS

TPW = 3136
TCH = 112
TNCH = TPW // TCH
TP = TPW * NW

_mesh = functools.partial(
    plsc.VectorSubcoreMesh,
    core_axis_name="c", subcore_axis_name="s",
    num_cores=NC, num_subcores=NS)


def _prep_edges_body(src, dst, et, fs, fd):
    fs[...] = et[...] * N + src[...]
    fd[...] = et[...] * N + dst[...]


def _prep_triples_body(srcp, relp, dstp, fsr, fdr):
    fsr[...] = relp[...] * N + srcp[...]
    fdr[...] = dstp[...] + R * N


def _mm_body(x, w, o):
    o[0] = jnp.dot(x[...], w[0], preferred_element_type=jnp.float32)


def _matmul(h, Wstk, d_in, d_out):
    nb = 10
    bs = N // nb
    return pl.pallas_call(
        _mm_body,
        grid=(R + 1, nb),
        in_specs=[
            pl.BlockSpec((bs, d_in), lambda r, i: (i, 0)),
            pl.BlockSpec((1, d_in, d_out), lambda r, i: (r, 0, 0)),
        ],
        out_specs=pl.BlockSpec((1, bs, d_out), lambda r, i: (r, i, 0)),
        out_shape=jax.ShapeDtypeStruct((R + 1, N, d_out), jnp.float32),
    )(h, Wstk)


def _deginv_body(d, o):
    o[...] = 1.0 / jnp.maximum(d[0] + d[1], 1.0)


def _finish1_body(a0, a1, slf, b, o):
    o[...] = jnp.maximum(a0[...] + a1[...] + slf[...] + b[...], 0.0)


def _finish2_body(a0, a1, slf, b, rd, o):
    h2 = a0[...] + a1[...] + slf[...] + b[...]
    o[R] = h2
    for r in range(R):
        o[r] = h2 * rd[r]


def _deg_body(fd_hbm, zeros_hbm, out_hbm, fd_v, ones_v, deg_sh):
    c = lax.axis_index("c")
    s = lax.axis_index("s")
    wid = c * NS + s
    pltpu.sync_copy(zeros_hbm.at[pl.ds(s * DEG_SL, DEG_SL)],
                    deg_sh.at[pl.ds(s * DEG_SL, DEG_SL)])
    for g in range(ECH // L):
        ones_v[pl.ds(g * L, L)] = jnp.ones((L,), jnp.float32)
    pltpu.sync_copy(fd_hbm.at[wid], fd_v)
    plsc.subcore_barrier()

    def body(j, carry):
        pltpu.sync_copy(ones_v, deg_sh.at[fd_v.at[j]], add=True)
        return carry

    lax.fori_loop(0, NCH, body, 0)
    plsc.subcore_barrier()
    pltpu.sync_copy(deg_sh.at[pl.ds(s * DEG_SL, DEG_SL)],
                    out_hbm.at[pl.ds(c * RNP + s * DEG_SL, DEG_SL)])


def _deg_kernel(fd, zeros_deg):
    k = pl.kernel(
        _deg_body,
        out_type=jax.ShapeDtypeStruct((NC * RNP,), jnp.float32),
        mesh=_mesh(),
        compiler_params=pltpu.CompilerParams(
            needs_layout_passes=False, use_tc_tiling_on_sc=False),
        scratch_types=[
            pltpu.VMEM((NCH, ECH), jnp.int32),
            pltpu.VMEM((ECH,), jnp.float32),
            pltpu.VMEM_SHARED((RNP,), jnp.float32),
        ],
    )
    return k(fd, zeros_deg)


def _layer_body(D, compute_norm, *refs):
    if compute_norm:
        (table_hbm, fs_hbm, dst_hbm, fd_hbm, dinv_hbm, zeros_hbm,
         out_hbm, norm_out_hbm,
         fs_v, dst_v, norm_v, fd_v, rows_v, dinv_sh, agg_sh) = refs
    else:
        (table_hbm, fs_hbm, dst_hbm, norm_hbm, zeros_hbm,
         out_hbm,
         fs_v, dst_v, norm_c, rows_v, agg_sh) = refs
    c = lax.axis_index("c")
    s = lax.axis_index("s")
    wid = c * NS + s
    pltpu.sync_copy(zeros_hbm.at[pl.ds(s * NP_SL, NP_SL)],
                    agg_sh.at[pl.ds(s * NP_SL, NP_SL)])
    pltpu.sync_copy(fs_hbm.at[wid], fs_v)
    pltpu.sync_copy(dst_hbm.at[wid], dst_v)
    if compute_norm:
        pltpu.sync_copy(fd_hbm.at[wid], fd_v)
        pltpu.sync_copy(dinv_hbm.at[pl.ds(s * DEG_SL, DEG_SL)],
                        dinv_sh.at[pl.ds(s * DEG_SL, DEG_SL)])
        plsc.subcore_barrier()

        def nbody(j, carry):
            pltpu.sync_copy(dinv_sh.at[fd_v.at[j]], norm_v.at[j])
            return carry

        lax.fori_loop(0, NCH, nbody, 0)
        pltpu.sync_copy(norm_v, norm_out_hbm.at[wid])
    plsc.subcore_barrier()

    def ebody(j, carry):
        if not compute_norm:
            pltpu.sync_copy(norm_hbm.at[wid].at[j], norm_c)
        pltpu.sync_copy(table_hbm.at[fs_v.at[j]], rows_v)

        def rbody(i, icarry):
            if compute_norm:
                sv = plsc.load_gather(
                    norm_v, [jnp.full((L,), j, jnp.int32),
                             jnp.full((L,), i, jnp.int32)])
            else:
                sv = plsc.load_gather(
                    norm_c, [jnp.full((L,), i, jnp.int32)])
            for g in range(D // L):
                rows_v[i, pl.ds(g * L, L)] = (
                    rows_v[i, pl.ds(g * L, L)] * sv)
            return icarry

        lax.fori_loop(0, ECH, rbody, 0)
        pltpu.sync_copy(rows_v, agg_sh.at[dst_v.at[j]], add=True)
        return carry

    lax.fori_loop(0, NCH, ebody, 0)
    plsc.subcore_barrier()
    pltpu.sync_copy(agg_sh.at[pl.ds(s * NP_SL, NP_SL)],
                    out_hbm.at[c, pl.ds(s * NP_SL, NP_SL)])


def _layer1_kernel(table, fs, dst2, fd, dinv, zeros_np):
    k = pl.kernel(
        functools.partial(_layer_body, D_MID, True),
        out_type=(
            jax.ShapeDtypeStruct((NC, NP, D_MID), jnp.float32),
            jax.ShapeDtypeStruct((NW, NCH, ECH), jnp.float32),
        ),
        mesh=_mesh(),
        compiler_params=pltpu.CompilerParams(
            needs_layout_passes=False, use_tc_tiling_on_sc=False),
        scratch_types=[
            pltpu.VMEM((NCH, ECH), jnp.int32),
            pltpu.VMEM((NCH, ECH), jnp.int32),
            pltpu.VMEM((NCH, ECH), jnp.float32),
            pltpu.VMEM((NCH, ECH), jnp.int32),
            pltpu.VMEM((ECH, D_MID), jnp.float32),
            pltpu.VMEM_SHARED((RNP,), jnp.float32),
            pltpu.VMEM_SHARED((NP, D_MID), jnp.float32),
        ],
    )
    return k(table, fs, dst2, fd, dinv, zeros_np)


def _layer2_kernel(table, fs, dst2, norm, zeros_np):
    k = pl.kernel(
        functools.partial(_layer_body, D_LAT, False),
        out_type=jax.ShapeDtypeStruct((NC, NP, D_LAT), jnp.float32),
        mesh=_mesh(),
        compiler_params=pltpu.CompilerParams(needs_layout_passes=False),
        scratch_types=[
            pltpu.VMEM((NCH, ECH), jnp.int32),
            pltpu.VMEM((NCH, ECH), jnp.int32),
            pltpu.VMEM((ECH,), jnp.float32),
            pltpu.VMEM((ECH, D_LAT), jnp.float32),
            pltpu.VMEM_SHARED((NP, D_LAT), jnp.float32),
        ],
    )
    return k(table, fs, dst2, norm, zeros_np)


def _decoder_body(table_hbm, fsr_hbm, fdr_hbm, out_hbm,
                  fsr_v, fdr_v, srows_v, drows_v, scores_v):
    c = lax.axis_index("c")
    s = lax.axis_index("s")
    wid = c * NS + s
    pltpu.sync_copy(fsr_hbm.at[wid], fsr_v)
    pltpu.sync_copy(fdr_hbm.at[wid], fdr_v)

    def jbody(j, carry):
        pltpu.sync_copy(table_hbm.at[fsr_v.at[j]], srows_v)
        pltpu.sync_copy(table_hbm.at[fdr_v.at[j]], drows_v)

        lanes = jnp.arange(L, dtype=jnp.int32)

        def gbody(tg, icarry):
            base = tg * L
            score = jnp.zeros((L,), jnp.float32)
            for u in range(L):
                t = base + u
                acc = srows_v[t, pl.ds(0, L)] * drows_v[t, pl.ds(0, L)]
                for g in range(1, D_LAT // L):
                    acc = acc + (srows_v[t, pl.ds(g * L, L)]
                                 * drows_v[t, pl.ds(g * L, L)])
                score = jnp.where(lanes == u, jnp.sum(acc), score)
            scores_v[j, pl.ds(base, L)] = score
            return icarry

        lax.fori_loop(0, TCH // L, gbody, 0)
        return carry

    lax.fori_loop(0, TNCH, jbody, 0)
    pltpu.sync_copy(scores_v, out_hbm.at[wid])


def _decoder_kernel(table, fsr, fdr):
    k = pl.kernel(
        _decoder_body,
        out_type=jax.ShapeDtypeStruct((NW, TNCH, TCH), jnp.float32),
        mesh=_mesh(),
        compiler_params=pltpu.CompilerParams(needs_layout_passes=False),
        scratch_types=[
            pltpu.VMEM((TNCH, TCH), jnp.int32),
            pltpu.VMEM((TNCH, TCH), jnp.int32),
            pltpu.VMEM((TCH, D_LAT), jnp.float32),
            pltpu.VMEM((TCH, D_LAT), jnp.float32),
            pltpu.VMEM((TNCH, TCH), jnp.float32),
        ],
    )
    return k(table, fsr, fdr)


def kernel(node_features, edge_index, edge_type, src_idx, rel_idx, dst_idx,
           W1, W1_self, b1, W2, W2_self, b2, Rdiag):
    f32 = jnp.float32
    src = edge_index[0].reshape(2500, 128)
    dst = edge_index[1].reshape(2500, 128)
    et = edge_type.reshape(2500, 128)

    fs, fd = pl.pallas_call(
        _prep_edges_body,
        out_shape=(jax.ShapeDtypeStruct((2500, 128), jnp.int32),
                   jax.ShapeDtypeStruct((2500, 128), jnp.int32)),
    )(src, dst, et)
    fs = fs.reshape(NW, NCH, ECH)
    fd = fd.reshape(NW, NCH, ECH)
    dst2 = edge_index[1].reshape(NW, NCH, ECH)

    padn = TP - T
    pad_iota = (jnp.arange(padn, dtype=jnp.int32) * 37) % N
    srcp = jnp.concatenate([src_idx, pad_iota]).reshape(TP // 128, 128)
    relp = jnp.concatenate(
        [rel_idx, jnp.zeros((padn,), jnp.int32)]).reshape(TP // 128, 128)
    dstp = jnp.concatenate([dst_idx, pad_iota]).reshape(TP // 128, 128)
    fsr, fdr = pl.pallas_call(
        _prep_triples_body,
        out_shape=(jax.ShapeDtypeStruct((TP // 128, 128), jnp.int32),
                   jax.ShapeDtypeStruct((TP // 128, 128), jnp.int32)),
    )(srcp, relp, dstp)
    fsr = fsr.reshape(NW, TNCH, TCH)
    fdr = fdr.reshape(NW, TNCH, TCH)

    zeros_deg = jnp.zeros((RNP,), f32)
    zeros_n1 = jnp.zeros((NP, D_MID), f32)
    zeros_n2 = jnp.zeros((NP, D_LAT), f32)

    deg_parts = _deg_kernel(fd, zeros_deg)
    dinv = pl.pallas_call(
        _deginv_body,
        out_shape=jax.ShapeDtypeStruct((RNP // 128, 128), f32),
    )(deg_parts.reshape(NC, RNP // 128, 128))
    dinv = dinv.reshape(RNP)

    W1a = jnp.concatenate([W1, W1_self[None]], axis=0)
    T1 = _matmul(node_features, W1a, D_IN, D_MID)
    table1 = T1.reshape((R + 1) * N, D_MID)
    agg1, norm = _layer1_kernel(table1, fs, dst2, fd, dinv, zeros_n1)

    nb = 10
    bs = N // nb
    H1 = pl.pallas_call(
        _finish1_body,
        grid=(nb,),
        in_specs=[
            pl.BlockSpec((bs, D_MID), lambda i: (i, 0)),
            pl.BlockSpec((bs, D_MID), lambda i: (i, 0)),
            pl.BlockSpec((bs, D_MID), lambda i: (i, 0)),
            pl.BlockSpec((1, D_MID), lambda i: (0, 0)),
        ],
        out_specs=pl.BlockSpec((bs, D_MID), lambda i: (i, 0)),
        out_shape=jax.ShapeDtypeStruct((N, D_MID), f32),
    )(agg1[0, :N], agg1[1, :N], T1[R], b1.reshape(1, D_MID))

    W2a = jnp.concatenate([W2, W2_self[None]], axis=0)
    T2 = _matmul(H1, W2a, D_MID, D_LAT)
    table2 = T2.reshape((R + 1) * N, D_LAT)
    agg2 = _layer2_kernel(table2, fs, dst2, norm, zeros_n2)

    tab_dec = pl.pallas_call(
        _finish2_body,
        grid=(nb,),
        in_specs=[
            pl.BlockSpec((bs, D_LAT), lambda i: (i, 0)),
            pl.BlockSpec((bs, D_LAT), lambda i: (i, 0)),
            pl.BlockSpec((bs, D_LAT), lambda i: (i, 0)),
            pl.BlockSpec((1, D_LAT), lambda i: (0, 0)),
            pl.BlockSpec((R, D_LAT), lambda i: (0, 0)),
        ],
        out_specs=pl.BlockSpec((R + 1, bs, D_LAT), lambda i: (0, i, 0)),
        out_shape=jax.ShapeDtypeStruct((R + 1, N, D_LAT), f32),
    )(agg2[0, :N], agg2[1, :N], T2[R], b2.reshape(1, D_LAT), Rdiag)

    scores = _decoder_kernel(tab_dec.reshape((R + 1) * N, D_LAT), fsr, fdr)
    return scores.reshape(TP)[:T]


# speedup vs baseline: 18.3158x; 1.0680x over previous
import functools

import jax
import jax.numpy as jnp
from jax import lax
from jax.experimental import pallas as pl
from jax.experimental.pallas import tpu as pltpu
from jax.experimental.pallas import tpu_sc as plsc

N = 10000
E = 320000
R = 4
D_IN = 128
D_MID = 64
D_LAT = 128
T = 100000

NC = 2
NS = 16
L = 16
NW = NC * NS

EPW = E // NW
ECH = 80
NCH = EPW // ECH

RN = R * N
RNP = 49152
DEG_SL = RNP // NS

NP = 10112
NP_SL = NP // NS

TPW = 3136
TCH = 112
TNCH = TPW // TCH
TP = TPW * NW

_mesh = functools.partial(
    plsc.VectorSubcoreMesh,
    core_axis_name="c", subcore_axis_name="s",
    num_cores=NC, num_subcores=NS)


def _prep_edges_body(src, dst, et, fs, fd):
    fs[...] = et[...] * N + src[...]
    fd[...] = et[...] * N + dst[...]


def _prep_triples_body(srcp, relp, dstp, fsr, fdr):
    fsr[...] = relp[...] * N + srcp[...]
    fdr[...] = dstp[...] + R * N


def _mm_body(x, w, o):
    o[0] = jnp.dot(x[...], w[0], preferred_element_type=jnp.float32)


def _matmul(h, Wstk, d_in, d_out):
    nb = 10
    bs = N // nb
    return pl.pallas_call(
        _mm_body,
        grid=(R + 1, nb),
        in_specs=[
            pl.BlockSpec((bs, d_in), lambda r, i: (i, 0)),
            pl.BlockSpec((1, d_in, d_out), lambda r, i: (r, 0, 0)),
        ],
        out_specs=pl.BlockSpec((1, bs, d_out), lambda r, i: (r, i, 0)),
        out_shape=jax.ShapeDtypeStruct((R + 1, N, d_out), jnp.float32),
    )(h, Wstk)


def _deginv_body(d, o):
    o[...] = 1.0 / jnp.maximum(d[0] + d[1], 1.0)


def _finish1_body(a0, a1, slf, b, o):
    o[...] = jnp.maximum(a0[...] + a1[...] + slf[...] + b[...], 0.0)


def _finish2_body(a0, a1, slf, b, rd, o):
    h2 = a0[...] + a1[...] + slf[...] + b[...]
    o[R] = h2
    for r in range(R):
        o[r] = h2 * rd[r]


def _deg_body(fd_hbm, zeros_hbm, out_hbm, fd_v, ones_v, deg_sh):
    c = lax.axis_index("c")
    s = lax.axis_index("s")
    wid = c * NS + s
    pltpu.sync_copy(zeros_hbm.at[pl.ds(s * DEG_SL, DEG_SL)],
                    deg_sh.at[pl.ds(s * DEG_SL, DEG_SL)])
    for g in range(ECH // L):
        ones_v[pl.ds(g * L, L)] = jnp.ones((L,), jnp.float32)
    pltpu.sync_copy(fd_hbm.at[wid], fd_v)
    plsc.subcore_barrier()

    def body(j, carry):
        pltpu.sync_copy(ones_v, deg_sh.at[fd_v.at[j]], add=True)
        return carry

    lax.fori_loop(0, NCH, body, 0)
    plsc.subcore_barrier()
    pltpu.sync_copy(deg_sh.at[pl.ds(s * DEG_SL, DEG_SL)],
                    out_hbm.at[pl.ds(c * RNP + s * DEG_SL, DEG_SL)])


def _deg_kernel(fd, zeros_deg):
    k = pl.kernel(
        _deg_body,
        out_type=jax.ShapeDtypeStruct((NC * RNP,), jnp.float32),
        mesh=_mesh(),
        compiler_params=pltpu.CompilerParams(
            needs_layout_passes=False, use_tc_tiling_on_sc=False),
        scratch_types=[
            pltpu.VMEM((NCH, ECH), jnp.int32),
            pltpu.VMEM((ECH,), jnp.float32),
            pltpu.VMEM_SHARED((RNP,), jnp.float32),
        ],
    )
    return k(fd, zeros_deg)


def _layer_body(D, compute_norm, *refs):
    if compute_norm:
        (table_hbm, fs_hbm, dst_hbm, fd_hbm, dinv_hbm, zeros_hbm,
         out_hbm, norm_out_hbm,
         fs_v, dst_v, norm_v, fd_v, rows_v, dinv_sh, agg_sh) = refs
    else:
        (table_hbm, fs_hbm, dst_hbm, norm_hbm, zeros_hbm,
         out_hbm,
         fs_v, dst_v, norm_c, rows_v, agg_sh) = refs
    c = lax.axis_index("c")
    s = lax.axis_index("s")
    wid = c * NS + s
    pltpu.sync_copy(zeros_hbm.at[pl.ds(s * NP_SL, NP_SL)],
                    agg_sh.at[pl.ds(s * NP_SL, NP_SL)])
    pltpu.sync_copy(fs_hbm.at[wid], fs_v)
    pltpu.sync_copy(dst_hbm.at[wid], dst_v)
    if compute_norm:
        pltpu.sync_copy(fd_hbm.at[wid], fd_v)
        pltpu.sync_copy(dinv_hbm.at[pl.ds(s * DEG_SL, DEG_SL)],
                        dinv_sh.at[pl.ds(s * DEG_SL, DEG_SL)])
        plsc.subcore_barrier()

        def nbody(j, carry):
            pltpu.sync_copy(dinv_sh.at[fd_v.at[j]], norm_v.at[j])
            return carry

        lax.fori_loop(0, NCH, nbody, 0)
        pltpu.sync_copy(norm_v, norm_out_hbm.at[wid])
    plsc.subcore_barrier()

    def ebody(j, carry):
        if not compute_norm:
            pltpu.sync_copy(norm_hbm.at[wid].at[j], norm_c)
        pltpu.sync_copy(table_hbm.at[fs_v.at[j]], rows_v)

        def rbody(i, icarry):
            if compute_norm:
                sv = plsc.load_gather(
                    norm_v, [jnp.full((L,), j, jnp.int32),
                             jnp.full((L,), i, jnp.int32)])
            else:
                sv = plsc.load_gather(
                    norm_c, [jnp.full((L,), i, jnp.int32)])
            for g in range(D // L):
                rows_v[i, pl.ds(g * L, L)] = (
                    rows_v[i, pl.ds(g * L, L)] * sv)
            return icarry

        lax.fori_loop(0, ECH, rbody, 0)
        pltpu.sync_copy(rows_v, agg_sh.at[dst_v.at[j]], add=True)
        return carry

    lax.fori_loop(0, NCH, ebody, 0)
    plsc.subcore_barrier()
    pltpu.sync_copy(agg_sh.at[pl.ds(s * NP_SL, NP_SL)],
                    out_hbm.at[c, pl.ds(s * NP_SL, NP_SL)])


def _layer1_kernel(table, fs, dst2, fd, dinv, zeros_np):
    k = pl.kernel(
        functools.partial(_layer_body, D_MID, True),
        out_type=(
            jax.ShapeDtypeStruct((NC, NP, D_MID), jnp.float32),
            jax.ShapeDtypeStruct((NW, NCH, ECH), jnp.float32),
        ),
        mesh=_mesh(),
        compiler_params=pltpu.CompilerParams(
            needs_layout_passes=False, use_tc_tiling_on_sc=False),
        scratch_types=[
            pltpu.VMEM((NCH, ECH), jnp.int32),
            pltpu.VMEM((NCH, ECH), jnp.int32),
            pltpu.VMEM((NCH, ECH), jnp.float32),
            pltpu.VMEM((NCH, ECH), jnp.int32),
            pltpu.VMEM((ECH, D_MID), jnp.float32),
            pltpu.VMEM_SHARED((RNP,), jnp.float32),
            pltpu.VMEM_SHARED((NP, D_MID), jnp.float32),
        ],
    )
    return k(table, fs, dst2, fd, dinv, zeros_np)


def _layer2_kernel(table, fs, dst2, norm, zeros_np):
    k = pl.kernel(
        functools.partial(_layer_body, D_LAT, False),
        out_type=jax.ShapeDtypeStruct((NC, NP, D_LAT), jnp.float32),
        mesh=_mesh(),
        compiler_params=pltpu.CompilerParams(needs_layout_passes=False),
        scratch_types=[
            pltpu.VMEM((NCH, ECH), jnp.int32),
            pltpu.VMEM((NCH, ECH), jnp.int32),
            pltpu.VMEM((ECH,), jnp.float32),
            pltpu.VMEM((ECH, D_LAT), jnp.float32),
            pltpu.VMEM_SHARED((NP, D_LAT), jnp.float32),
        ],
    )
    return k(table, fs, dst2, norm, zeros_np)


def _decoder_body(table_hbm, fsr_hbm, fdr_hbm, out_hbm,
                  fsr_v, fdr_v, srows_v, drows_v, scores_v, sem0, sem1):
    c = lax.axis_index("c")
    s = lax.axis_index("s")
    wid = c * NS + s
    pltpu.sync_copy(fsr_hbm.at[wid], fsr_v)
    pltpu.sync_copy(fdr_hbm.at[wid], fdr_v)
    sems = (sem0, sem1)

    def fire(j, b):
        pltpu.async_copy(table_hbm.at[fsr_v.at[j]], srows_v.at[b], sems[b])
        pltpu.async_copy(table_hbm.at[fdr_v.at[j]], drows_v.at[b], sems[b])

    def drain(j, b):
        pltpu.make_async_copy(
            table_hbm.at[fsr_v.at[j]], srows_v.at[b], sems[b]).wait()
        pltpu.make_async_copy(
            table_hbm.at[fdr_v.at[j]], drows_v.at[b], sems[b]).wait()

    def compute(j, b):
        lanes = jnp.arange(L, dtype=jnp.int32)

        def gbody(tg, icarry):
            base = tg * L
            score = jnp.zeros((L,), jnp.float32)
            for u in range(L):
                t = base + u
                acc = (srows_v[b, t, pl.ds(0, L)]
                       * drows_v[b, t, pl.ds(0, L)])
                for g in range(1, D_LAT // L):
                    acc = acc + (srows_v[b, t, pl.ds(g * L, L)]
                                 * drows_v[b, t, pl.ds(g * L, L)])
                score = jnp.where(lanes == u, jnp.sum(acc), score)
            scores_v[j, pl.ds(base, L)] = score
            return icarry

        lax.fori_loop(0, TCH // L, gbody, 0)

    fire(0, 0)

    def pair(jj, carry):
        j0 = 2 * jj
        j1 = j0 + 1
        fire(j1, 1)
        drain(j0, 0)
        compute(j0, 0)

        @pl.when(jj < TNCH // 2 - 1)
        def _():
            fire(j0 + 2, 0)

        drain(j1, 1)
        compute(j1, 1)
        return carry

    lax.fori_loop(0, TNCH // 2, pair, 0)
    pltpu.sync_copy(scores_v, out_hbm.at[wid])


def _decoder_kernel(table, fsr, fdr):
    k = pl.kernel(
        _decoder_body,
        out_type=jax.ShapeDtypeStruct((NW, TNCH, TCH), jnp.float32),
        mesh=_mesh(),
        compiler_params=pltpu.CompilerParams(needs_layout_passes=False),
        scratch_types=[
            pltpu.VMEM((TNCH, TCH), jnp.int32),
            pltpu.VMEM((TNCH, TCH), jnp.int32),
            pltpu.VMEM((2, TCH, D_LAT), jnp.float32),
            pltpu.VMEM((2, TCH, D_LAT), jnp.float32),
            pltpu.VMEM((TNCH, TCH), jnp.float32),
            pltpu.SemaphoreType.DMA,
            pltpu.SemaphoreType.DMA,
        ],
    )
    return k(table, fsr, fdr)


def kernel(node_features, edge_index, edge_type, src_idx, rel_idx, dst_idx,
           W1, W1_self, b1, W2, W2_self, b2, Rdiag):
    f32 = jnp.float32
    src = edge_index[0].reshape(2500, 128)
    dst = edge_index[1].reshape(2500, 128)
    et = edge_type.reshape(2500, 128)

    fs, fd = pl.pallas_call(
        _prep_edges_body,
        out_shape=(jax.ShapeDtypeStruct((2500, 128), jnp.int32),
                   jax.ShapeDtypeStruct((2500, 128), jnp.int32)),
    )(src, dst, et)
    fs = fs.reshape(NW, NCH, ECH)
    fd = fd.reshape(NW, NCH, ECH)
    dst2 = edge_index[1].reshape(NW, NCH, ECH)

    padn = TP - T
    pad_iota = (jnp.arange(padn, dtype=jnp.int32) * 37) % N
    srcp = jnp.concatenate([src_idx, pad_iota]).reshape(TP // 128, 128)
    relp = jnp.concatenate(
        [rel_idx, jnp.zeros((padn,), jnp.int32)]).reshape(TP // 128, 128)
    dstp = jnp.concatenate([dst_idx, pad_iota]).reshape(TP // 128, 128)
    fsr, fdr = pl.pallas_call(
        _prep_triples_body,
        out_shape=(jax.ShapeDtypeStruct((TP // 128, 128), jnp.int32),
                   jax.ShapeDtypeStruct((TP // 128, 128), jnp.int32)),
    )(srcp, relp, dstp)
    fsr = fsr.reshape(NW, TNCH, TCH)
    fdr = fdr.reshape(NW, TNCH, TCH)

    zeros_deg = jnp.zeros((RNP,), f32)
    zeros_n1 = jnp.zeros((NP, D_MID), f32)
    zeros_n2 = jnp.zeros((NP, D_LAT), f32)

    deg_parts = _deg_kernel(fd, zeros_deg)
    dinv = pl.pallas_call(
        _deginv_body,
        out_shape=jax.ShapeDtypeStruct((RNP // 128, 128), f32),
    )(deg_parts.reshape(NC, RNP // 128, 128))
    dinv = dinv.reshape(RNP)

    W1a = jnp.concatenate([W1, W1_self[None]], axis=0)
    T1 = _matmul(node_features, W1a, D_IN, D_MID)
    table1 = T1.reshape((R + 1) * N, D_MID)
    agg1, norm = _layer1_kernel(table1, fs, dst2, fd, dinv, zeros_n1)

    nb = 10
    bs = N // nb
    H1 = pl.pallas_call(
        _finish1_body,
        grid=(nb,),
        in_specs=[
            pl.BlockSpec((bs, D_MID), lambda i: (i, 0)),
            pl.BlockSpec((bs, D_MID), lambda i: (i, 0)),
            pl.BlockSpec((bs, D_MID), lambda i: (i, 0)),
            pl.BlockSpec((1, D_MID), lambda i: (0, 0)),
        ],
        out_specs=pl.BlockSpec((bs, D_MID), lambda i: (i, 0)),
        out_shape=jax.ShapeDtypeStruct((N, D_MID), f32),
    )(agg1[0, :N], agg1[1, :N], T1[R], b1.reshape(1, D_MID))

    W2a = jnp.concatenate([W2, W2_self[None]], axis=0)
    T2 = _matmul(H1, W2a, D_MID, D_LAT)
    table2 = T2.reshape((R + 1) * N, D_LAT)
    agg2 = _layer2_kernel(table2, fs, dst2, norm, zeros_n2)

    tab_dec = pl.pallas_call(
        _finish2_body,
        grid=(nb,),
        in_specs=[
            pl.BlockSpec((bs, D_LAT), lambda i: (i, 0)),
            pl.BlockSpec((bs, D_LAT), lambda i: (i, 0)),
            pl.BlockSpec((bs, D_LAT), lambda i: (i, 0)),
            pl.BlockSpec((1, D_LAT), lambda i: (0, 0)),
            pl.BlockSpec((R, D_LAT), lambda i: (0, 0)),
        ],
        out_specs=pl.BlockSpec((R + 1, bs, D_LAT), lambda i: (0, i, 0)),
        out_shape=jax.ShapeDtypeStruct((R + 1, N, D_LAT), f32),
    )(agg2[0, :N], agg2[1, :N], T2[R], b2.reshape(1, D_LAT), Rdiag)

    scores = _decoder_kernel(tab_dec.reshape((R + 1) * N, D_LAT), fsr, fdr)
    return scores.reshape(TP)[:T]


# speedup vs baseline: 20.4923x; 1.1188x over previous
import functools

import jax
import jax.numpy as jnp
from jax import lax
from jax.experimental import pallas as pl
from jax.experimental.pallas import tpu as pltpu
from jax.experimental.pallas import tpu_sc as plsc

N = 10000
E = 320000
R = 4
D_IN = 128
D_MID = 64
D_LAT = 128
T = 100000

NC = 2
NS = 16
L = 16
NW = NC * NS

EPW = E // NW
ECH = 80
NCH = EPW // ECH
ECHD = 80
NCHD = EPW // ECHD

RN = R * N
RNP = 49152
DEG_SL = RNP // NS

NP = 10112
NP_SL = NP // NS

TPW = 3136
TCH = 112
TNCH = TPW // TCH
TP = TPW * NW

_mesh = functools.partial(
    plsc.VectorSubcoreMesh,
    core_axis_name="c", subcore_axis_name="s",
    num_cores=NC, num_subcores=NS)


def _prep_edges_body(src, dst, et, fs, fd):
    fs[...] = et[...] * N + src[...]
    fd[...] = et[...] * N + dst[...]


def _prep_triples_body(srcp, relp, dstp, fsr, fdr):
    fsr[...] = relp[...] * N + srcp[...]
    fdr[...] = dstp[...] + R * N


def _mm_body(x, w, o):
    o[0] = jnp.dot(x[...], w[0], preferred_element_type=jnp.float32)


def _matmul(h, Wstk, d_in, d_out):
    nb = 10
    bs = N // nb
    return pl.pallas_call(
        _mm_body,
        grid=(R + 1, nb),
        in_specs=[
            pl.BlockSpec((bs, d_in), lambda r, i: (i, 0)),
            pl.BlockSpec((1, d_in, d_out), lambda r, i: (r, 0, 0)),
        ],
        out_specs=pl.BlockSpec((1, bs, d_out), lambda r, i: (r, i, 0)),
        out_shape=jax.ShapeDtypeStruct((R + 1, N, d_out), jnp.float32),
    )(h, Wstk)


def _deginv_body(d, o):
    o[...] = 1.0 / jnp.maximum(d[0] + d[1], 1.0)


def _finish1_body(a0, a1, slf, b, o):
    o[...] = jnp.maximum(a0[...] + a1[...] + slf[...] + b[...], 0.0)


def _finish2_body(a0, a1, slf, b, rd, o):
    h2 = a0[...] + a1[...] + slf[...] + b[...]
    o[R] = h2
    for r in range(R):
        o[r] = h2 * rd[r]


def _deg_body(fd_hbm, zeros_hbm, out_hbm, fd_v, ones_v, deg_sh):
    c = lax.axis_index("c")
    s = lax.axis_index("s")
    wid = c * NS + s
    pltpu.sync_copy(zeros_hbm.at[pl.ds(s * DEG_SL, DEG_SL)],
                    deg_sh.at[pl.ds(s * DEG_SL, DEG_SL)])
    for g in range(ECHD // L):
        ones_v[pl.ds(g * L, L)] = jnp.ones((L,), jnp.float32)
    pltpu.sync_copy(fd_hbm.at[wid], fd_v)
    plsc.subcore_barrier()

    def body(j, carry):
        pltpu.sync_copy(ones_v, deg_sh.at[fd_v.at[j]], add=True)
        return carry

    lax.fori_loop(0, NCHD, body, 0)
    plsc.subcore_barrier()
    pltpu.sync_copy(deg_sh.at[pl.ds(s * DEG_SL, DEG_SL)],
                    out_hbm.at[pl.ds(c * RNP + s * DEG_SL, DEG_SL)])


def _deg_kernel(fd, zeros_deg):
    k = pl.kernel(
        _deg_body,
        out_type=jax.ShapeDtypeStruct((NC * RNP,), jnp.float32),
        mesh=_mesh(),
        compiler_params=pltpu.CompilerParams(
            needs_layout_passes=False, use_tc_tiling_on_sc=False),
        scratch_types=[
            pltpu.VMEM((NCHD, ECHD), jnp.int32),
            pltpu.VMEM((ECHD,), jnp.float32),
            pltpu.VMEM_SHARED((RNP,), jnp.float32),
        ],
    )
    return k(fd, zeros_deg)


def _layer_body(D, compute_norm, *refs):
    if compute_norm:
        (table_hbm, fs_hbm, dst_hbm, fd_hbm, dinv_hbm, zeros_hbm,
         out_hbm, norm_out_hbm,
         fs_v, dst_v, norm_v, fd_v, rows_v, dinv_sh, agg_sh,
         sem0, sem1) = refs
    else:
        (table_hbm, fs_hbm, dst_hbm, norm_hbm, zeros_hbm,
         out_hbm,
         fs_v, dst_v, norm_c, rows_v, agg_sh) = refs
    c = lax.axis_index("c")
    s = lax.axis_index("s")
    wid = c * NS + s
    pltpu.sync_copy(zeros_hbm.at[pl.ds(s * NP_SL, NP_SL)],
                    agg_sh.at[pl.ds(s * NP_SL, NP_SL)])
    pltpu.sync_copy(fs_hbm.at[wid], fs_v)
    pltpu.sync_copy(dst_hbm.at[wid], dst_v)
    if compute_norm:
        pltpu.sync_copy(fd_hbm.at[wid], fd_v)
        pltpu.sync_copy(dinv_hbm.at[pl.ds(s * DEG_SL, DEG_SL)],
                        dinv_sh.at[pl.ds(s * DEG_SL, DEG_SL)])
        plsc.subcore_barrier()

        def nbody(j, carry):
            pltpu.sync_copy(dinv_sh.at[fd_v.at[j]], norm_v.at[j])
            return carry

        lax.fori_loop(0, NCH, nbody, 0)
        pltpu.sync_copy(norm_v, norm_out_hbm.at[wid])
    plsc.subcore_barrier()

    if compute_norm:
        sems = (sem0, sem1)

        def fire(j, b):
            pltpu.async_copy(
                table_hbm.at[fs_v.at[j]], rows_v.at[b], sems[b])

        def drain(j, b):
            pltpu.make_async_copy(
                table_hbm.at[fs_v.at[j]], rows_v.at[b], sems[b]).wait()

        def compute(j, b):
            def rbody(i, icarry):
                sv = plsc.load_gather(
                    norm_v, [jnp.full((L,), j, jnp.int32),
                             jnp.full((L,), i, jnp.int32)])
                for g in range(D // L):
                    rows_v[b, i, pl.ds(g * L, L)] = (
                        rows_v[b, i, pl.ds(g * L, L)] * sv)
                return icarry

            lax.fori_loop(0, ECH, rbody, 0)
            pltpu.sync_copy(rows_v.at[b], agg_sh.at[dst_v.at[j]], add=True)

        fire(0, 0)

        def pair(jj, carry):
            j0 = 2 * jj
            j1 = j0 + 1
            fire(j1, 1)
            drain(j0, 0)
            compute(j0, 0)
            fire(j0 + 2, 0)
            drain(j1, 1)
            compute(j1, 1)
            return carry

        lax.fori_loop(0, NCH // 2, pair, 0)
        drain(NCH - 1, 0)
        compute(NCH - 1, 0)
    else:
        def ebody(j, carry):
            pltpu.sync_copy(norm_hbm.at[wid].at[j], norm_c)
            pltpu.sync_copy(table_hbm.at[fs_v.at[j]], rows_v)

            def rbody(i, icarry):
                sv = plsc.load_gather(
                    norm_c, [jnp.full((L,), i, jnp.int32)])
                for g in range(D // L):
                    rows_v[i, pl.ds(g * L, L)] = (
                        rows_v[i, pl.ds(g * L, L)] * sv)
                return icarry

            lax.fori_loop(0, ECH, rbody, 0)
            pltpu.sync_copy(rows_v, agg_sh.at[dst_v.at[j]], add=True)
            return carry

        lax.fori_loop(0, NCH, ebody, 0)
    plsc.subcore_barrier()
    pltpu.sync_copy(agg_sh.at[pl.ds(s * NP_SL, NP_SL)],
                    out_hbm.at[c, pl.ds(s * NP_SL, NP_SL)])


def _layer1_kernel(table, fs, dst2, fd, dinv, zeros_np):
    k = pl.kernel(
        functools.partial(_layer_body, D_MID, True),
        out_type=(
            jax.ShapeDtypeStruct((NC, NP, D_MID), jnp.float32),
            jax.ShapeDtypeStruct((NW, NCH, ECH), jnp.float32),
        ),
        mesh=_mesh(),
        compiler_params=pltpu.CompilerParams(
            needs_layout_passes=False, use_tc_tiling_on_sc=False),
        scratch_types=[
            pltpu.VMEM((NCH, ECH), jnp.int32),
            pltpu.VMEM((NCH, ECH), jnp.int32),
            pltpu.VMEM((NCH, ECH), jnp.float32),
            pltpu.VMEM((NCH, ECH), jnp.int32),
            pltpu.VMEM((2, ECH, D_MID), jnp.float32),
            pltpu.VMEM_SHARED((RNP,), jnp.float32),
            pltpu.VMEM_SHARED((NP, D_MID), jnp.float32),
            pltpu.SemaphoreType.DMA,
            pltpu.SemaphoreType.DMA,
        ],
    )
    return k(table, fs, dst2, fd, dinv, zeros_np)


def _layer2_kernel(table, fs, dst2, norm, zeros_np):
    k = pl.kernel(
        functools.partial(_layer_body, D_LAT, False),
        out_type=jax.ShapeDtypeStruct((NC, NP, D_LAT), jnp.float32),
        mesh=_mesh(),
        compiler_params=pltpu.CompilerParams(needs_layout_passes=False),
        scratch_types=[
            pltpu.VMEM((NCH, ECH), jnp.int32),
            pltpu.VMEM((NCH, ECH), jnp.int32),
            pltpu.VMEM((ECH,), jnp.float32),
            pltpu.VMEM((ECH, D_LAT), jnp.float32),
            pltpu.VMEM_SHARED((NP, D_LAT), jnp.float32),
        ],
    )
    return k(table, fs, dst2, norm, zeros_np)


def _decoder_body(table_hbm, fsr_hbm, fdr_hbm, out_hbm,
                  fsr_v, fdr_v, srows_v, drows_v, scores_v, sem0, sem1):
    c = lax.axis_index("c")
    s = lax.axis_index("s")
    wid = c * NS + s
    pltpu.sync_copy(fsr_hbm.at[wid], fsr_v)
    pltpu.sync_copy(fdr_hbm.at[wid], fdr_v)
    sems = (sem0, sem1)

    def fire(j, b):
        pltpu.async_copy(table_hbm.at[fsr_v.at[j]], srows_v.at[b], sems[b])
        pltpu.async_copy(table_hbm.at[fdr_v.at[j]], drows_v.at[b], sems[b])

    def drain(j, b):
        pltpu.make_async_copy(
            table_hbm.at[fsr_v.at[j]], srows_v.at[b], sems[b]).wait()
        pltpu.make_async_copy(
            table_hbm.at[fdr_v.at[j]], drows_v.at[b], sems[b]).wait()

    def compute(j, b):
        lanes = jnp.arange(L, dtype=jnp.int32)

        def gbody(tg, icarry):
            base = tg * L
            score = jnp.zeros((L,), jnp.float32)
            for u in range(L):
                t = base + u
                acc = (srows_v[b, t, pl.ds(0, L)]
                       * drows_v[b, t, pl.ds(0, L)])
                for g in range(1, D_LAT // L):
                    acc = acc + (srows_v[b, t, pl.ds(g * L, L)]
                                 * drows_v[b, t, pl.ds(g * L, L)])
                score = jnp.where(lanes == u, jnp.sum(acc), score)
            scores_v[j, pl.ds(base, L)] = score
            return icarry

        lax.fori_loop(0, TCH // L, gbody, 0)

    fire(0, 0)

    def pair(jj, carry):
        j0 = 2 * jj
        j1 = j0 + 1
        fire(j1, 1)
        drain(j0, 0)
        compute(j0, 0)

        @pl.when(jj < TNCH // 2 - 1)
        def _():
            fire(j0 + 2, 0)

        drain(j1, 1)
        compute(j1, 1)
        return carry

    lax.fori_loop(0, TNCH // 2, pair, 0)
    pltpu.sync_copy(scores_v, out_hbm.at[wid])


def _decoder_kernel(table, fsr, fdr):
    k = pl.kernel(
        _decoder_body,
        out_type=jax.ShapeDtypeStruct((NW, TNCH, TCH), jnp.float32),
        mesh=_mesh(),
        compiler_params=pltpu.CompilerParams(needs_layout_passes=False),
        scratch_types=[
            pltpu.VMEM((TNCH, TCH), jnp.int32),
            pltpu.VMEM((TNCH, TCH), jnp.int32),
            pltpu.VMEM((2, TCH, D_LAT), jnp.float32),
            pltpu.VMEM((2, TCH, D_LAT), jnp.float32),
            pltpu.VMEM((TNCH, TCH), jnp.float32),
            pltpu.SemaphoreType.DMA,
            pltpu.SemaphoreType.DMA,
        ],
    )
    return k(table, fsr, fdr)


def kernel(node_features, edge_index, edge_type, src_idx, rel_idx, dst_idx,
           W1, W1_self, b1, W2, W2_self, b2, Rdiag):
    f32 = jnp.float32
    src = edge_index[0].reshape(2500, 128)
    dst = edge_index[1].reshape(2500, 128)
    et = edge_type.reshape(2500, 128)

    fs, fd = pl.pallas_call(
        _prep_edges_body,
        out_shape=(jax.ShapeDtypeStruct((2500, 128), jnp.int32),
                   jax.ShapeDtypeStruct((2500, 128), jnp.int32)),
    )(src, dst, et)
    fs = fs.reshape(NW, NCH, ECH)
    fd = fd.reshape(NW, NCH, ECH)
    dst2 = edge_index[1].reshape(NW, NCH, ECH)

    padn = TP - T
    pad_iota = (jnp.arange(padn, dtype=jnp.int32) * 37) % N
    srcp = jnp.concatenate([src_idx, pad_iota]).reshape(TP // 128, 128)
    relp = jnp.concatenate(
        [rel_idx, jnp.zeros((padn,), jnp.int32)]).reshape(TP // 128, 128)
    dstp = jnp.concatenate([dst_idx, pad_iota]).reshape(TP // 128, 128)
    fsr, fdr = pl.pallas_call(
        _prep_triples_body,
        out_shape=(jax.ShapeDtypeStruct((TP // 128, 128), jnp.int32),
                   jax.ShapeDtypeStruct((TP // 128, 128), jnp.int32)),
    )(srcp, relp, dstp)
    fsr = fsr.reshape(NW, TNCH, TCH)
    fdr = fdr.reshape(NW, TNCH, TCH)

    zeros_deg = jnp.zeros((RNP,), f32)
    zeros_n1 = jnp.zeros((NP, D_MID), f32)
    zeros_n2 = jnp.zeros((NP, D_LAT), f32)

    deg_parts = _deg_kernel(fd.reshape(NW, NCHD, ECHD), zeros_deg)
    dinv = pl.pallas_call(
        _deginv_body,
        out_shape=jax.ShapeDtypeStruct((RNP // 128, 128), f32),
    )(deg_parts.reshape(NC, RNP // 128, 128))
    dinv = dinv.reshape(RNP)

    W1a = jnp.concatenate([W1, W1_self[None]], axis=0)
    T1 = _matmul(node_features, W1a, D_IN, D_MID)
    table1 = T1.reshape((R + 1) * N, D_MID)
    agg1, norm = _layer1_kernel(table1, fs, dst2, fd, dinv, zeros_n1)

    nb = 10
    bs = N // nb
    H1 = pl.pallas_call(
        _finish1_body,
        grid=(nb,),
        in_specs=[
            pl.BlockSpec((bs, D_MID), lambda i: (i, 0)),
            pl.BlockSpec((bs, D_MID), lambda i: (i, 0)),
            pl.BlockSpec((bs, D_MID), lambda i: (i, 0)),
            pl.BlockSpec((1, D_MID), lambda i: (0, 0)),
        ],
        out_specs=pl.BlockSpec((bs, D_MID), lambda i: (i, 0)),
        out_shape=jax.ShapeDtypeStruct((N, D_MID), f32),
    )(agg1[0, :N], agg1[1, :N], T1[R], b1.reshape(1, D_MID))

    W2a = jnp.concatenate([W2, W2_self[None]], axis=0)
    T2 = _matmul(H1, W2a, D_MID, D_LAT)
    table2 = T2.reshape((R + 1) * N, D_LAT)
    agg2 = _layer2_kernel(table2, fs, dst2, norm, zeros_n2)

    tab_dec = pl.pallas_call(
        _finish2_body,
        grid=(nb,),
        in_specs=[
            pl.BlockSpec((bs, D_LAT), lambda i: (i, 0)),
            pl.BlockSpec((bs, D_LAT), lambda i: (i, 0)),
            pl.BlockSpec((bs, D_LAT), lambda i: (i, 0)),
            pl.BlockSpec((1, D_LAT), lambda i: (0, 0)),
            pl.BlockSpec((R, D_LAT), lambda i: (0, 0)),
        ],
        out_specs=pl.BlockSpec((R + 1, bs, D_LAT), lambda i: (0, i, 0)),
        out_shape=jax.ShapeDtypeStruct((R + 1, N, D_LAT), f32),
    )(agg2[0, :N], agg2[1, :N], T2[R], b2.reshape(1, D_LAT), Rdiag)

    scores = _decoder_kernel(tab_dec.reshape((R + 1) * N, D_LAT), fsr, fdr)
    return scores.reshape(TP)[:T]


# speedup vs baseline: 24.6079x; 1.2008x over previous
import functools

import jax
import jax.numpy as jnp
from jax import lax
from jax.experimental import pallas as pl
from jax.experimental.pallas import tpu as pltpu
from jax.experimental.pallas import tpu_sc as plsc

N = 10000
E = 320000
R = 4
D_IN = 128
D_MID = 64
D_LAT = 128
T = 100000

NC = 2
NS = 16
L = 16
NW = NC * NS

EPW = E // NW
ECH = 80
NCH = EPW // ECH
ECHD = 80
NCHD = EPW // ECHD

RN = R * N
RNP = 49152
DEG_SL = RNP // NS

NP = 10112
NP_SL = NP // NS

TPW = 3136
TCH = 112
TNCH = TPW // TCH
TP = TPW * NW

_mesh = functools.partial(
    plsc.VectorSubcoreMesh,
    core_axis_name="c", subcore_axis_name="s",
    num_cores=NC, num_subcores=NS)


def _prep_edges_body(src, dst, et, fs, fd):
    fs[...] = et[...] * N + src[...]
    fd[...] = et[...] * N + dst[...]


def _prep_triples_body(srcp, relp, dstp, fsr, fdr):
    fsr[...] = relp[...] * N + srcp[...]
    fdr[...] = dstp[...] + R * N


def _mm_body(x, w, o):
    o[0] = jnp.dot(x[...], w[0], preferred_element_type=jnp.float32)


def _matmul(h, Wstk, d_in, d_out):
    nb = 10
    bs = N // nb
    return pl.pallas_call(
        _mm_body,
        grid=(R + 1, nb),
        in_specs=[
            pl.BlockSpec((bs, d_in), lambda r, i: (i, 0)),
            pl.BlockSpec((1, d_in, d_out), lambda r, i: (r, 0, 0)),
        ],
        out_specs=pl.BlockSpec((1, bs, d_out), lambda r, i: (r, i, 0)),
        out_shape=jax.ShapeDtypeStruct((R + 1, N, d_out), jnp.float32),
    )(h, Wstk)


def _deginv_body(d, o):
    o[...] = 1.0 / jnp.maximum(d[0] + d[1], 1.0)


def _finish1_body(a0, a1, slf, b, o):
    o[...] = jnp.maximum(a0[...] + a1[...] + slf[...] + b[...], 0.0)


def _finish2_body(a0, a1, slf, b, rd, o):
    h2 = a0[...] + a1[...] + slf[...] + b[...]
    o[R] = h2
    for r in range(R):
        o[r] = h2 * rd[r]


def _deg_body(fd_hbm, zeros_hbm, out_hbm, fd_v, ones_v, deg_sh):
    c = lax.axis_index("c")
    s = lax.axis_index("s")
    wid = c * NS + s
    pltpu.sync_copy(zeros_hbm.at[pl.ds(s * DEG_SL, DEG_SL)],
                    deg_sh.at[pl.ds(s * DEG_SL, DEG_SL)])
    for g in range(ECHD // L):
        ones_v[pl.ds(g * L, L)] = jnp.ones((L,), jnp.float32)
    pltpu.sync_copy(fd_hbm.at[wid], fd_v)
    plsc.subcore_barrier()

    def body(j, carry):
        pltpu.sync_copy(ones_v, deg_sh.at[fd_v.at[j]], add=True)
        return carry

    lax.fori_loop(0, NCHD, body, 0)
    plsc.subcore_barrier()
    pltpu.sync_copy(deg_sh.at[pl.ds(s * DEG_SL, DEG_SL)],
                    out_hbm.at[pl.ds(c * RNP + s * DEG_SL, DEG_SL)])


def _deg_kernel(fd, zeros_deg):
    k = pl.kernel(
        _deg_body,
        out_type=jax.ShapeDtypeStruct((NC * RNP,), jnp.float32),
        mesh=_mesh(),
        compiler_params=pltpu.CompilerParams(
            needs_layout_passes=False, use_tc_tiling_on_sc=False),
        scratch_types=[
            pltpu.VMEM((NCHD, ECHD), jnp.int32),
            pltpu.VMEM((ECHD,), jnp.float32),
            pltpu.VMEM_SHARED((RNP,), jnp.float32),
        ],
    )
    return k(fd, zeros_deg)


def _layer_body(D, compute_norm, *refs):
    if compute_norm:
        (table_hbm, fs_hbm, dst_hbm, fd_hbm, dinv_hbm, zeros_hbm,
         out_hbm, norm_out_hbm,
         fs_v, dst_v, norm_v, fd_v, rows_v, dinv_sh, agg_sh,
         sem0, sem1) = refs
    else:
        (table_hbm, fs_hbm, dst_hbm, norm_hbm, zeros_hbm,
         out_hbm,
         fs_v, dst_v, norm_c, rows_v, agg_sh, sem0, sem1) = refs
    c = lax.axis_index("c")
    s = lax.axis_index("s")
    wid = c * NS + s
    pltpu.sync_copy(zeros_hbm.at[pl.ds(s * NP_SL, NP_SL)],
                    agg_sh.at[pl.ds(s * NP_SL, NP_SL)])
    pltpu.sync_copy(fs_hbm.at[wid], fs_v)
    pltpu.sync_copy(dst_hbm.at[wid], dst_v)
    if compute_norm:
        pltpu.sync_copy(fd_hbm.at[wid], fd_v)
        pltpu.sync_copy(dinv_hbm.at[pl.ds(s * DEG_SL, DEG_SL)],
                        dinv_sh.at[pl.ds(s * DEG_SL, DEG_SL)])
        plsc.subcore_barrier()

        def nbody(j, carry):
            pltpu.sync_copy(dinv_sh.at[fd_v.at[j]], norm_v.at[j])
            return carry

        lax.fori_loop(0, NCH, nbody, 0)
        pltpu.sync_copy(norm_v, norm_out_hbm.at[wid])
    plsc.subcore_barrier()
    sems = (sem0, sem1)

    def fire(j, b):
        pltpu.async_copy(table_hbm.at[fs_v.at[j]], rows_v.at[b], sems[b])

    def drain(j, b):
        pltpu.make_async_copy(
            table_hbm.at[fs_v.at[j]], rows_v.at[b], sems[b]).wait()

    def compute(j, b):
        if not compute_norm:
            pltpu.sync_copy(norm_hbm.at[wid].at[j], norm_c)

        def rbody(i, icarry):
            if compute_norm:
                sv = plsc.load_gather(
                    norm_v, [jnp.full((L,), j, jnp.int32),
                             jnp.full((L,), i, jnp.int32)])
            else:
                sv = plsc.load_gather(
                    norm_c, [jnp.full((L,), i, jnp.int32)])
            for g in range(D // L):
                rows_v[b, i, pl.ds(g * L, L)] = (
                    rows_v[b, i, pl.ds(g * L, L)] * sv)
            return icarry

        lax.fori_loop(0, ECH, rbody, 0)
        pltpu.sync_copy(rows_v.at[b], agg_sh.at[dst_v.at[j]], add=True)

    fire(0, 0)

    def pair(jj, carry):
        j0 = 2 * jj
        j1 = j0 + 1
        fire(j1, 1)
        drain(j0, 0)
        compute(j0, 0)
        fire(j0 + 2, 0)
        drain(j1, 1)
        compute(j1, 1)
        return carry

    lax.fori_loop(0, NCH // 2, pair, 0)
    drain(NCH - 1, 0)
    compute(NCH - 1, 0)
    plsc.subcore_barrier()
    pltpu.sync_copy(agg_sh.at[pl.ds(s * NP_SL, NP_SL)],
                    out_hbm.at[c, pl.ds(s * NP_SL, NP_SL)])


def _layer1_kernel(table, fs, dst2, fd, dinv, zeros_np):
    k = pl.kernel(
        functools.partial(_layer_body, D_MID, True),
        out_type=(
            jax.ShapeDtypeStruct((NC, NP, D_MID), jnp.float32),
            jax.ShapeDtypeStruct((NW, NCH, ECH), jnp.float32),
        ),
        mesh=_mesh(),
        compiler_params=pltpu.CompilerParams(
            needs_layout_passes=False, use_tc_tiling_on_sc=False),
        scratch_types=[
            pltpu.VMEM((NCH, ECH), jnp.int32),
            pltpu.VMEM((NCH, ECH), jnp.int32),
            pltpu.VMEM((NCH, ECH), jnp.float32),
            pltpu.VMEM((NCH, ECH), jnp.int32),
            pltpu.VMEM((2, ECH, D_MID), jnp.float32),
            pltpu.VMEM_SHARED((RNP,), jnp.float32),
            pltpu.VMEM_SHARED((NP, D_MID), jnp.float32),
            pltpu.SemaphoreType.DMA,
            pltpu.SemaphoreType.DMA,
        ],
    )
    return k(table, fs, dst2, fd, dinv, zeros_np)


def _layer2_kernel(table, fs, dst2, norm, zeros_np):
    k = pl.kernel(
        functools.partial(_layer_body, D_LAT, False),
        out_type=jax.ShapeDtypeStruct((NC, NP, D_LAT), jnp.float32),
        mesh=_mesh(),
        compiler_params=pltpu.CompilerParams(
            needs_layout_passes=False, use_tc_tiling_on_sc=False),
        scratch_types=[
            pltpu.VMEM((NCH, ECH), jnp.int32),
            pltpu.VMEM((NCH, ECH), jnp.int32),
            pltpu.VMEM((ECH,), jnp.float32),
            pltpu.VMEM((2, ECH, D_LAT), jnp.float32),
            pltpu.VMEM_SHARED((NP, D_LAT), jnp.float32),
            pltpu.SemaphoreType.DMA,
            pltpu.SemaphoreType.DMA,
        ],
    )
    return k(table, fs, dst2, norm, zeros_np)


def _decoder_body(table_hbm, fsr_hbm, fdr_hbm, out_hbm,
                  fsr_v, fdr_v, srows_v, drows_v, scores_v, sem0, sem1):
    c = lax.axis_index("c")
    s = lax.axis_index("s")
    wid = c * NS + s
    pltpu.sync_copy(fsr_hbm.at[wid], fsr_v)
    pltpu.sync_copy(fdr_hbm.at[wid], fdr_v)
    sems = (sem0, sem1)

    def fire(j, b):
        pltpu.async_copy(table_hbm.at[fsr_v.at[j]], srows_v.at[b], sems[b])
        pltpu.async_copy(table_hbm.at[fdr_v.at[j]], drows_v.at[b], sems[b])

    def drain(j, b):
        pltpu.make_async_copy(
            table_hbm.at[fsr_v.at[j]], srows_v.at[b], sems[b]).wait()
        pltpu.make_async_copy(
            table_hbm.at[fdr_v.at[j]], drows_v.at[b], sems[b]).wait()

    def compute(j, b):
        lanes = jnp.arange(L, dtype=jnp.int32)

        def gbody(tg, icarry):
            base = tg * L
            score = jnp.zeros((L,), jnp.float32)
            for u in range(L):
                t = base + u
                acc = (srows_v[b, t, pl.ds(0, L)]
                       * drows_v[b, t, pl.ds(0, L)])
                for g in range(1, D_LAT // L):
                    acc = acc + (srows_v[b, t, pl.ds(g * L, L)]
                                 * drows_v[b, t, pl.ds(g * L, L)])
                score = jnp.where(lanes == u, jnp.sum(acc), score)
            scores_v[j, pl.ds(base, L)] = score
            return icarry

        lax.fori_loop(0, TCH // L, gbody, 0)

    fire(0, 0)

    def pair(jj, carry):
        j0 = 2 * jj
        j1 = j0 + 1
        fire(j1, 1)
        drain(j0, 0)
        compute(j0, 0)

        @pl.when(jj < TNCH // 2 - 1)
        def _():
            fire(j0 + 2, 0)

        drain(j1, 1)
        compute(j1, 1)
        return carry

    lax.fori_loop(0, TNCH // 2, pair, 0)
    pltpu.sync_copy(scores_v, out_hbm.at[wid])


def _decoder_kernel(table, fsr, fdr):
    k = pl.kernel(
        _decoder_body,
        out_type=jax.ShapeDtypeStruct((NW, TNCH, TCH), jnp.float32),
        mesh=_mesh(),
        compiler_params=pltpu.CompilerParams(needs_layout_passes=False),
        scratch_types=[
            pltpu.VMEM((TNCH, TCH), jnp.int32),
            pltpu.VMEM((TNCH, TCH), jnp.int32),
            pltpu.VMEM((2, TCH, D_LAT), jnp.float32),
            pltpu.VMEM((2, TCH, D_LAT), jnp.float32),
            pltpu.VMEM((TNCH, TCH), jnp.float32),
            pltpu.SemaphoreType.DMA,
            pltpu.SemaphoreType.DMA,
        ],
    )
    return k(table, fsr, fdr)


def kernel(node_features, edge_index, edge_type, src_idx, rel_idx, dst_idx,
           W1, W1_self, b1, W2, W2_self, b2, Rdiag):
    f32 = jnp.float32
    src = edge_index[0].reshape(2500, 128)
    dst = edge_index[1].reshape(2500, 128)
    et = edge_type.reshape(2500, 128)

    fs, fd = pl.pallas_call(
        _prep_edges_body,
        out_shape=(jax.ShapeDtypeStruct((2500, 128), jnp.int32),
                   jax.ShapeDtypeStruct((2500, 128), jnp.int32)),
    )(src, dst, et)
    fs = fs.reshape(NW, NCH, ECH)
    fd = fd.reshape(NW, NCH, ECH)
    dst2 = edge_index[1].reshape(NW, NCH, ECH)

    padn = TP - T
    pad_iota = (jnp.arange(padn, dtype=jnp.int32) * 37) % N
    srcp = jnp.concatenate([src_idx, pad_iota]).reshape(TP // 128, 128)
    relp = jnp.concatenate(
        [rel_idx, jnp.zeros((padn,), jnp.int32)]).reshape(TP // 128, 128)
    dstp = jnp.concatenate([dst_idx, pad_iota]).reshape(TP // 128, 128)
    fsr, fdr = pl.pallas_call(
        _prep_triples_body,
        out_shape=(jax.ShapeDtypeStruct((TP // 128, 128), jnp.int32),
                   jax.ShapeDtypeStruct((TP // 128, 128), jnp.int32)),
    )(srcp, relp, dstp)
    fsr = fsr.reshape(NW, TNCH, TCH)
    fdr = fdr.reshape(NW, TNCH, TCH)

    zeros_deg = jnp.zeros((RNP,), f32)
    zeros_n1 = jnp.zeros((NP, D_MID), f32)
    zeros_n2 = jnp.zeros((NP, D_LAT), f32)

    deg_parts = _deg_kernel(fd.reshape(NW, NCHD, ECHD), zeros_deg)
    dinv = pl.pallas_call(
        _deginv_body,
        out_shape=jax.ShapeDtypeStruct((RNP // 128, 128), f32),
    )(deg_parts.reshape(NC, RNP // 128, 128))
    dinv = dinv.reshape(RNP)

    W1a = jnp.concatenate([W1, W1_self[None]], axis=0)
    T1 = _matmul(node_features, W1a, D_IN, D_MID)
    table1 = T1.reshape((R + 1) * N, D_MID)
    agg1, norm = _layer1_kernel(table1, fs, dst2, fd, dinv, zeros_n1)

    nb = 10
    bs = N // nb
    H1 = pl.pallas_call(
        _finish1_body,
        grid=(nb,),
        in_specs=[
            pl.BlockSpec((bs, D_MID), lambda i: (i, 0)),
            pl.BlockSpec((bs, D_MID), lambda i: (i, 0)),
            pl.BlockSpec((bs, D_MID), lambda i: (i, 0)),
            pl.BlockSpec((1, D_MID), lambda i: (0, 0)),
        ],
        out_specs=pl.BlockSpec((bs, D_MID), lambda i: (i, 0)),
        out_shape=jax.ShapeDtypeStruct((N, D_MID), f32),
    )(agg1[0, :N], agg1[1, :N], T1[R], b1.reshape(1, D_MID))

    W2a = jnp.concatenate([W2, W2_self[None]], axis=0)
    T2 = _matmul(H1, W2a, D_MID, D_LAT)
    table2 = T2.reshape((R + 1) * N, D_LAT)
    agg2 = _layer2_kernel(table2, fs, dst2, norm, zeros_n2)

    tab_dec = pl.pallas_call(
        _finish2_body,
        grid=(nb,),
        in_specs=[
            pl.BlockSpec((bs, D_LAT), lambda i: (i, 0)),
            pl.BlockSpec((bs, D_LAT), lambda i: (i, 0)),
            pl.BlockSpec((bs, D_LAT), lambda i: (i, 0)),
            pl.BlockSpec((1, D_LAT), lambda i: (0, 0)),
            pl.BlockSpec((R, D_LAT), lambda i: (0, 0)),
        ],
        out_specs=pl.BlockSpec((R + 1, bs, D_LAT), lambda i: (0, i, 0)),
        out_shape=jax.ShapeDtypeStruct((R + 1, N, D_LAT), f32),
    )(agg2[0, :N], agg2[1, :N], T2[R], b2.reshape(1, D_LAT), Rdiag)

    scores = _decoder_kernel(tab_dec.reshape((R + 1) * N, D_LAT), fsr, fdr)
    return scores.reshape(TP)[:T]


# speedup vs baseline: 26.6967x; 1.0849x over previous
import functools

import jax
import jax.numpy as jnp
from jax import lax
from jax.experimental import pallas as pl
from jax.experimental.pallas import tpu as pltpu
from jax.experimental.pallas import tpu_sc as plsc

N = 10000
E = 320000
R = 4
D_IN = 128
D_MID = 64
D_LAT = 128
T = 100000

NC = 2
NS = 16
L = 16
NW = NC * NS

EPW = E // NW
ECH = 80
NCH = EPW // ECH
ECHD = 80
NCHD = EPW // ECHD

RN = R * N
RNP = 49152
DEG_SL = RNP // NS

NP = 10112
NP_SL = NP // NS

TPW = 3136
TCH = 112
TNCH = TPW // TCH
TP = TPW * NW

_mesh = functools.partial(
    plsc.VectorSubcoreMesh,
    core_axis_name="c", subcore_axis_name="s",
    num_cores=NC, num_subcores=NS)


def _prep_edges_body(src, dst, et, fs, fd):
    fs[...] = et[...] * N + src[...]
    fd[...] = et[...] * N + dst[...]


def _prep_triples_body(srcp, relp, dstp, fsr, fdr):
    fsr[...] = relp[...] * N + srcp[...]
    fdr[...] = dstp[...] + R * N


def _mm_body(x, w, o):
    o[0] = jnp.dot(x[...], w[0], preferred_element_type=jnp.float32)


def _matmul(h, Wstk, d_in, d_out):
    nb = 10
    bs = N // nb
    return pl.pallas_call(
        _mm_body,
        grid=(R + 1, nb),
        in_specs=[
            pl.BlockSpec((bs, d_in), lambda r, i: (i, 0)),
            pl.BlockSpec((1, d_in, d_out), lambda r, i: (r, 0, 0)),
        ],
        out_specs=pl.BlockSpec((1, bs, d_out), lambda r, i: (r, i, 0)),
        out_shape=jax.ShapeDtypeStruct((R + 1, N, d_out), jnp.float32),
    )(h, Wstk)


def _deginv_body(d, o):
    o[...] = 1.0 / jnp.maximum(d[0] + d[1], 1.0)


def _finish1_body(a0, a1, slf, b, o):
    o[...] = jnp.maximum(a0[...] + a1[...] + slf[...] + b[...], 0.0)


def _finish2_body(a0, a1, slf, b, rd, o):
    h2 = a0[...] + a1[...] + slf[...] + b[...]
    o[R] = h2
    for r in range(R):
        o[r] = h2 * rd[r]


def _deg_body(fd_hbm, zeros_hbm, out_hbm, fd_v, ones_v, deg_sh):
    c = lax.axis_index("c")
    s = lax.axis_index("s")
    wid = c * NS + s
    pltpu.sync_copy(zeros_hbm.at[pl.ds(s * DEG_SL, DEG_SL)],
                    deg_sh.at[pl.ds(s * DEG_SL, DEG_SL)])
    for g in range(ECHD // L):
        ones_v[pl.ds(g * L, L)] = jnp.ones((L,), jnp.float32)
    pltpu.sync_copy(fd_hbm.at[wid], fd_v)
    plsc.subcore_barrier()

    def body(j, carry):
        pltpu.sync_copy(ones_v, deg_sh.at[fd_v.at[j]], add=True)
        return carry

    lax.fori_loop(0, NCHD, body, 0)
    plsc.subcore_barrier()
    pltpu.sync_copy(deg_sh.at[pl.ds(s * DEG_SL, DEG_SL)],
                    out_hbm.at[pl.ds(c * RNP + s * DEG_SL, DEG_SL)])


def _deg_kernel(fd, zeros_deg):
    k = pl.kernel(
        _deg_body,
        out_type=jax.ShapeDtypeStruct((NC * RNP,), jnp.float32),
        mesh=_mesh(),
        compiler_params=pltpu.CompilerParams(
            needs_layout_passes=False, use_tc_tiling_on_sc=False),
        scratch_types=[
            pltpu.VMEM((NCHD, ECHD), jnp.int32),
            pltpu.VMEM((ECHD,), jnp.float32),
            pltpu.VMEM_SHARED((RNP,), jnp.float32),
        ],
    )
    return k(fd, zeros_deg)


def _layer_body(D, compute_norm, *refs):
    if compute_norm:
        (table_hbm, fs_hbm, dst_hbm, fd_hbm, dinv_hbm, zeros_hbm,
         out_hbm, norm_out_hbm,
         fs_v, dst_v, norm_v, fd_v, rows_v, dinv_sh, agg_sh,
         sem0, sem1) = refs
    else:
        (table_hbm, fs_hbm, dst_hbm, norm_hbm, zeros_hbm,
         out_hbm,
         fs_v, dst_v, norm_c, rows_v, agg_sh, sem0, sem1) = refs
    c = lax.axis_index("c")
    s = lax.axis_index("s")
    wid = c * NS + s
    pltpu.sync_copy(zeros_hbm.at[pl.ds(s * NP_SL, NP_SL)],
                    agg_sh.at[pl.ds(s * NP_SL, NP_SL)])
    pltpu.sync_copy(fs_hbm.at[wid], fs_v)
    pltpu.sync_copy(dst_hbm.at[wid], dst_v)
    if compute_norm:
        pltpu.sync_copy(fd_hbm.at[wid], fd_v)
        pltpu.sync_copy(dinv_hbm.at[pl.ds(s * DEG_SL, DEG_SL)],
                        dinv_sh.at[pl.ds(s * DEG_SL, DEG_SL)])
        plsc.subcore_barrier()

        def nbody(j, carry):
            pltpu.sync_copy(dinv_sh.at[fd_v.at[j]], norm_v.at[j])
            return carry

        lax.fori_loop(0, NCH, nbody, 0)
        pltpu.sync_copy(norm_v, norm_out_hbm.at[wid])
    plsc.subcore_barrier()
    sems = (sem0, sem1)

    def fire(j, b):
        if not compute_norm:
            pltpu.async_copy(norm_hbm.at[wid].at[j], norm_c.at[b], sems[b])
        pltpu.async_copy(table_hbm.at[fs_v.at[j]], rows_v.at[b], sems[b])

    def drain(j, b):
        if not compute_norm:
            pltpu.make_async_copy(
                norm_hbm.at[wid].at[j], norm_c.at[b], sems[b]).wait()
        pltpu.make_async_copy(
            table_hbm.at[fs_v.at[j]], rows_v.at[b], sems[b]).wait()

    def compute(j, b):
        def rbody(i, icarry):
            if compute_norm:
                sv = plsc.load_gather(
                    norm_v, [jnp.full((L,), j, jnp.int32),
                             jnp.full((L,), i, jnp.int32)])
            else:
                sv = plsc.load_gather(
                    norm_c, [jnp.full((L,), b, jnp.int32),
                             jnp.full((L,), i, jnp.int32)])
            for g in range(D // L):
                rows_v[b, i, pl.ds(g * L, L)] = (
                    rows_v[b, i, pl.ds(g * L, L)] * sv)
            return icarry

        lax.fori_loop(0, ECH, rbody, 0)
        pltpu.sync_copy(rows_v.at[b], agg_sh.at[dst_v.at[j]], add=True)

    fire(0, 0)

    def pair(jj, carry):
        j0 = 2 * jj
        j1 = j0 + 1
        fire(j1, 1)
        drain(j0, 0)
        compute(j0, 0)
        fire(j0 + 2, 0)
        drain(j1, 1)
        compute(j1, 1)
        return carry

    lax.fori_loop(0, NCH // 2, pair, 0)
    drain(NCH - 1, 0)
    compute(NCH - 1, 0)
    plsc.subcore_barrier()
    pltpu.sync_copy(agg_sh.at[pl.ds(s * NP_SL, NP_SL)],
                    out_hbm.at[c, pl.ds(s * NP_SL, NP_SL)])


def _layer1_kernel(table, fs, dst2, fd, dinv, zeros_np):
    k = pl.kernel(
        functools.partial(_layer_body, D_MID, True),
        out_type=(
            jax.ShapeDtypeStruct((NC, NP, D_MID), jnp.float32),
            jax.ShapeDtypeStruct((NW, NCH, ECH), jnp.float32),
        ),
        mesh=_mesh(),
        compiler_params=pltpu.CompilerParams(
            needs_layout_passes=False, use_tc_tiling_on_sc=False),
        scratch_types=[
            pltpu.VMEM((NCH, ECH), jnp.int32),
            pltpu.VMEM((NCH, ECH), jnp.int32),
            pltpu.VMEM((NCH, ECH), jnp.float32),
            pltpu.VMEM((NCH, ECH), jnp.int32),
            pltpu.VMEM((2, ECH, D_MID), jnp.float32),
            pltpu.VMEM_SHARED((RNP,), jnp.float32),
            pltpu.VMEM_SHARED((NP, D_MID), jnp.float32),
            pltpu.SemaphoreType.DMA,
            pltpu.SemaphoreType.DMA,
        ],
    )
    return k(table, fs, dst2, fd, dinv, zeros_np)


def _layer2_kernel(table, fs, dst2, norm, zeros_np):
    k = pl.kernel(
        functools.partial(_layer_body, D_LAT, False),
        out_type=jax.ShapeDtypeStruct((NC, NP, D_LAT), jnp.float32),
        mesh=_mesh(),
        compiler_params=pltpu.CompilerParams(
            needs_layout_passes=False, use_tc_tiling_on_sc=False),
        scratch_types=[
            pltpu.VMEM((NCH, ECH), jnp.int32),
            pltpu.VMEM((NCH, ECH), jnp.int32),
            pltpu.VMEM((2, ECH), jnp.float32),
            pltpu.VMEM((2, ECH, D_LAT), jnp.float32),
            pltpu.VMEM_SHARED((NP, D_LAT), jnp.float32),
            pltpu.SemaphoreType.DMA,
            pltpu.SemaphoreType.DMA,
        ],
    )
    return k(table, fs, dst2, norm, zeros_np)


def _decoder_body(table_hbm, fsr_hbm, fdr_hbm, out_hbm,
                  fsr_v, fdr_v, srows_v, drows_v, scores_v, sem0, sem1):
    c = lax.axis_index("c")
    s = lax.axis_index("s")
    wid = c * NS + s
    pltpu.sync_copy(fsr_hbm.at[wid], fsr_v)
    pltpu.sync_copy(fdr_hbm.at[wid], fdr_v)
    sems = (sem0, sem1)

    def fire(j, b):
        pltpu.async_copy(table_hbm.at[fsr_v.at[j]], srows_v.at[b], sems[b])
        pltpu.async_copy(table_hbm.at[fdr_v.at[j]], drows_v.at[b], sems[b])

    def drain(j, b):
        pltpu.make_async_copy(
            table_hbm.at[fsr_v.at[j]], srows_v.at[b], sems[b]).wait()
        pltpu.make_async_copy(
            table_hbm.at[fdr_v.at[j]], drows_v.at[b], sems[b]).wait()

    def compute(j, b):
        lanes = jnp.arange(L, dtype=jnp.int32)

        def gbody(tg, icarry):
            base = tg * L
            score = jnp.zeros((L,), jnp.float32)
            for u in range(L):
                t = base + u
                acc = (srows_v[b, t, pl.ds(0, L)]
                       * drows_v[b, t, pl.ds(0, L)])
                for g in range(1, D_LAT // L):
                    acc = acc + (srows_v[b, t, pl.ds(g * L, L)]
                                 * drows_v[b, t, pl.ds(g * L, L)])
                score = jnp.where(lanes == u, jnp.sum(acc), score)
            scores_v[j, pl.ds(base, L)] = score
            return icarry

        lax.fori_loop(0, TCH // L, gbody, 0)

    fire(0, 0)

    def pair(jj, carry):
        j0 = 2 * jj
        j1 = j0 + 1
        fire(j1, 1)
        drain(j0, 0)
        compute(j0, 0)

        @pl.when(jj < TNCH // 2 - 1)
        def _():
            fire(j0 + 2, 0)

        drain(j1, 1)
        compute(j1, 1)
        return carry

    lax.fori_loop(0, TNCH // 2, pair, 0)
    pltpu.sync_copy(scores_v, out_hbm.at[wid])


def _decoder_kernel(table, fsr, fdr):
    k = pl.kernel(
        _decoder_body,
        out_type=jax.ShapeDtypeStruct((NW, TNCH, TCH), jnp.float32),
        mesh=_mesh(),
        compiler_params=pltpu.CompilerParams(needs_layout_passes=False),
        scratch_types=[
            pltpu.VMEM((TNCH, TCH), jnp.int32),
            pltpu.VMEM((TNCH, TCH), jnp.int32),
            pltpu.VMEM((2, TCH, D_LAT), jnp.float32),
            pltpu.VMEM((2, TCH, D_LAT), jnp.float32),
            pltpu.VMEM((TNCH, TCH), jnp.float32),
            pltpu.SemaphoreType.DMA,
            pltpu.SemaphoreType.DMA,
        ],
    )
    return k(table, fsr, fdr)


def kernel(node_features, edge_index, edge_type, src_idx, rel_idx, dst_idx,
           W1, W1_self, b1, W2, W2_self, b2, Rdiag):
    f32 = jnp.float32
    src = edge_index[0].reshape(2500, 128)
    dst = edge_index[1].reshape(2500, 128)
    et = edge_type.reshape(2500, 128)

    fs, fd = pl.pallas_call(
        _prep_edges_body,
        out_shape=(jax.ShapeDtypeStruct((2500, 128), jnp.int32),
                   jax.ShapeDtypeStruct((2500, 128), jnp.int32)),
    )(src, dst, et)
    fs = fs.reshape(NW, NCH, ECH)
    fd = fd.reshape(NW, NCH, ECH)
    dst2 = edge_index[1].reshape(NW, NCH, ECH)

    padn = TP - T
    pad_iota = (jnp.arange(padn, dtype=jnp.int32) * 37) % N
    srcp = jnp.concatenate([src_idx, pad_iota]).reshape(TP // 128, 128)
    relp = jnp.concatenate(
        [rel_idx, jnp.zeros((padn,), jnp.int32)]).reshape(TP // 128, 128)
    dstp = jnp.concatenate([dst_idx, pad_iota]).reshape(TP // 128, 128)
    fsr, fdr = pl.pallas_call(
        _prep_triples_body,
        out_shape=(jax.ShapeDtypeStruct((TP // 128, 128), jnp.int32),
                   jax.ShapeDtypeStruct((TP // 128, 128), jnp.int32)),
    )(srcp, relp, dstp)
    fsr = fsr.reshape(NW, TNCH, TCH)
    fdr = fdr.reshape(NW, TNCH, TCH)

    zeros_deg = jnp.zeros((RNP,), f32)
    zeros_n1 = jnp.zeros((NP, D_MID), f32)
    zeros_n2 = jnp.zeros((NP, D_LAT), f32)

    deg_parts = _deg_kernel(fd.reshape(NW, NCHD, ECHD), zeros_deg)
    dinv = pl.pallas_call(
        _deginv_body,
        out_shape=jax.ShapeDtypeStruct((RNP // 128, 128), f32),
    )(deg_parts.reshape(NC, RNP // 128, 128))
    dinv = dinv.reshape(RNP)

    W1a = jnp.concatenate([W1, W1_self[None]], axis=0)
    T1 = _matmul(node_features, W1a, D_IN, D_MID)
    table1 = T1.reshape((R + 1) * N, D_MID)
    agg1, norm = _layer1_kernel(table1, fs, dst2, fd, dinv, zeros_n1)

    nb = 10
    bs = N // nb
    H1 = pl.pallas_call(
        _finish1_body,
        grid=(nb,),
        in_specs=[
            pl.BlockSpec((bs, D_MID), lambda i: (i, 0)),
            pl.BlockSpec((bs, D_MID), lambda i: (i, 0)),
            pl.BlockSpec((bs, D_MID), lambda i: (i, 0)),
            pl.BlockSpec((1, D_MID), lambda i: (0, 0)),
        ],
        out_specs=pl.BlockSpec((bs, D_MID), lambda i: (i, 0)),
        out_shape=jax.ShapeDtypeStruct((N, D_MID), f32),
    )(agg1[0, :N], agg1[1, :N], T1[R], b1.reshape(1, D_MID))

    W2a = jnp.concatenate([W2, W2_self[None]], axis=0)
    T2 = _matmul(H1, W2a, D_MID, D_LAT)
    table2 = T2.reshape((R + 1) * N, D_LAT)
    agg2 = _layer2_kernel(table2, fs, dst2, norm, zeros_n2)

    tab_dec = pl.pallas_call(
        _finish2_body,
        grid=(nb,),
        in_specs=[
            pl.BlockSpec((bs, D_LAT), lambda i: (i, 0)),
            pl.BlockSpec((bs, D_LAT), lambda i: (i, 0)),
            pl.BlockSpec((bs, D_LAT), lambda i: (i, 0)),
            pl.BlockSpec((1, D_LAT), lambda i: (0, 0)),
            pl.BlockSpec((R, D_LAT), lambda i: (0, 0)),
        ],
        out_specs=pl.BlockSpec((R + 1, bs, D_LAT), lambda i: (0, i, 0)),
        out_shape=jax.ShapeDtypeStruct((R + 1, N, D_LAT), f32),
    )(agg2[0, :N], agg2[1, :N], T2[R], b2.reshape(1, D_LAT), Rdiag)

    scores = _decoder_kernel(tab_dec.reshape((R + 1) * N, D_LAT), fsr, fdr)
    return scores.reshape(TP)[:T]
